# Initial kernel scaffold; baseline (speedup 1.0000x reference)
import jax
import jax.numpy as jnp
from jax.experimental import pallas as pl


def kernel(x, edge_index, edge_attr, params):
    raise NotImplementedError("write your pallas kernel here")


# speedup vs baseline: 1.6042x; 1.6042x over previous
import functools

import jax
import jax.numpy as jnp
from jax import lax
from jax.experimental import pallas as pl
from jax.experimental.pallas import tpu as pltpu
from jax.experimental.pallas import tpu_sc as plsc

F = 64
NC = 2
NS = 16
C = 128
STEPS = 7


def _ceil_to(v, m):
    return ((v + m - 1) // m) * m


def _mm(a, b):
    return lax.dot_general(a, b, (((1,), (0,)), ((), ())),
                           preferred_element_type=jnp.float32)


def _full(shape):
    return pl.BlockSpec(shape, lambda i: (0,) * len(shape))


def _encode_nodes(x, wn1, bn1, wn2, bn2, wsd, wnenc, bnu1, *, bn):
    n = x.shape[0]

    def body(x_r, wn1_r, bn1_r, wn2_r, bn2_r, wsd_r, wnenc_r,
             bnu1_r, encx_o, ab_o, dx_o):
        h = jnp.maximum(x_r[...] * wn1_r[...] + bn1_r[...], 0.0)
        encx = jnp.maximum(_mm(h, wn2_r[...]) + bn2_r[...], 0.0)
        encx_o[...] = encx
        ab_o[...] = _mm(encx, wsd_r[...])
        dx_o[...] = _mm(encx, wnenc_r[...]) + bnu1_r[...]

    return pl.pallas_call(
        body,
        grid=(n // bn,),
        in_specs=[pl.BlockSpec((bn, 1), lambda i: (i, 0)),
                  _full((1, F)), _full((1, F)), _full((F, F)), _full((1, F)),
                  _full((F, 2 * F)), _full((F, F)), _full((1, F))],
        out_specs=[pl.BlockSpec((bn, F), lambda i: (i, 0)),
                   pl.BlockSpec((bn, 2 * F), lambda i: (i, 0)),
                   pl.BlockSpec((bn, F), lambda i: (i, 0))],
        out_shape=[jax.ShapeDtypeStruct((n, F), jnp.float32),
                   jax.ShapeDtypeStruct((n, 2 * F), jnp.float32),
                   jax.ShapeDtypeStruct((n, F), jnp.float32)],
    )(x, wn1, bn1, wn2, bn2, wsd, wnenc, bnu1)


def _encode_edges(ea, we1, be1, we2, be2, wee, beu1, *, be):
    e = ea.shape[0]

    def body(ea_r, we1_r, be1_r, we2_r, be2_r, wee_r, beu1_r, ec_o, cee_o):
        ea_b = ea_r[...]
        pre = (ea_b[:, 0:1] * we1_r[0:1, :] + ea_b[:, 1:2] * we1_r[1:2, :]
               + ea_b[:, 2:3] * we1_r[2:3, :] + be1_r[...])
        h = jnp.maximum(pre, 0.0)
        ee = jnp.maximum(_mm(h, we2_r[...]) + be2_r[...], 0.0)
        ec_o[...] = ee
        cee_o[...] = _mm(ee, wee_r[...]) + beu1_r[...]

    out = jax.ShapeDtypeStruct((e, F), jnp.float32)
    return pl.pallas_call(
        body,
        grid=(e // be,),
        in_specs=[pl.BlockSpec((be, 3), lambda i: (i, 0)),
                  _full((3, F)), _full((1, F)), _full((F, F)), _full((1, F)),
                  _full((F, F)), _full((1, F))],
        out_specs=[pl.BlockSpec((be, F), lambda i: (i, 0))] * 2,
        out_shape=[out] * 2,
    )(ea, we1, be1, we2, be2, wee, beu1)


def _edge_step(ga2, gb2, ec, cee, wec, we2, be2, *, be):
    e = ga2.shape[0]

    def body(ga_r, gb_r, ec_r, cee_r, wec_r, we2_r, be2_r, out_o):
        h1 = jnp.maximum(
            ga_r[:, 0:F] + gb_r[:, F:2 * F]
            + _mm(ec_r[...], wec_r[...]) + cee_r[...],
            0.0)
        out_o[...] = jnp.maximum(_mm(h1, we2_r[...]) + be2_r[...], 0.0)

    return pl.pallas_call(
        body,
        grid=(e // be,),
        in_specs=[pl.BlockSpec((be, 2 * F), lambda i: (i, 0)),
                  pl.BlockSpec((be, 2 * F), lambda i: (i, 0)),
                  pl.BlockSpec((be, F), lambda i: (i, 0)),
                  pl.BlockSpec((be, F), lambda i: (i, 0)),
                  _full((F, F)), _full((F, F)), _full((1, F))],
        out_specs=pl.BlockSpec((be, F), lambda i: (i, 0)),
        out_shape=jax.ShapeDtypeStruct((e, F), jnp.float32),
    )(ga2, gb2, ec, cee, wec, we2, be2)


def _node_step(recv2, xc, dx, wnx, wnr, wn2, bn2, wsd, *, bn):
    n = xc.shape[0]
    nh = n // 2
    bh = nh // bn

    def body(recv_r, xc_r, dx_r, wnx_r, wnr_r, wn2_r, bn2_r, wsd_r,
             xc_o, ab_o):
        h = jnp.maximum(
            _mm(xc_r[...], wnx_r[...]) + _mm(recv_r[0], wnr_r[...])
            + dx_r[...], 0.0)
        xc_n = jnp.maximum(_mm(h, wn2_r[...]) + bn2_r[...], 0.0)
        xc_o[...] = xc_n
        ab_o[...] = _mm(xc_n, wsd_r[...])

    return pl.pallas_call(
        body,
        grid=(n // bn,),
        in_specs=[pl.BlockSpec((1, bn, F), lambda i: (i // bh, i % bh, 0)),
                  pl.BlockSpec((bn, F), lambda i: (i, 0)),
                  pl.BlockSpec((bn, F), lambda i: (i, 0)),
                  _full((F, F)), _full((F, F)), _full((F, F)), _full((1, F)),
                  _full((F, 2 * F))],
        out_specs=[pl.BlockSpec((bn, F), lambda i: (i, 0)),
                   pl.BlockSpec((bn, 2 * F), lambda i: (i, 0))],
        out_shape=[jax.ShapeDtypeStruct((n, F), jnp.float32),
                   jax.ShapeDtypeStruct((n, 2 * F), jnp.float32)],
    )(recv2, xc, dx, wnx, wnr, wn2, bn2, wsd)


def _decode(xc, wd1, bd1, wd2, bd2, wd3, bd3, *, bn):
    n = xc.shape[0]

    def body(xc_r, wd1_r, bd1_r, wd2_r, bd2_r, wd3_r, bd3_r, out_o):
        h1 = jnp.maximum(_mm(xc_r[...], wd1_r[...]) + bd1_r[...], 0.0)
        h2 = jnp.maximum(_mm(h1, wd2_r[...]) + bd2_r[...], 0.0)
        out_o[...] = _mm(h2, wd3_r[...]) + bd3_r[...]

    return pl.pallas_call(
        body,
        grid=(n // bn,),
        in_specs=[pl.BlockSpec((bn, F), lambda i: (i, 0)),
                  _full((F, F)), _full((1, F)), _full((F, F)), _full((1, F)),
                  _full((F, 1)), _full((1, 1))],
        out_specs=pl.BlockSpec((bn, 1), lambda i: (i, 0)),
        out_shape=jax.ShapeDtypeStruct((n, 1), jnp.float32),
    )(xc, wd1, bd1, wd2, bd2, wd3, bd3)


@functools.lru_cache(maxsize=None)
def _make_gather(n, e):
    nw = NC * NS
    nchunk = e // C
    kg = _ceil_to(-(-nchunk // nw), 8)
    mesh = plsc.VectorSubcoreMesh(core_axis_name="c", subcore_axis_name="s")

    @functools.partial(
        pl.kernel,
        out_type=[jax.ShapeDtypeStruct((e, 2 * F), jnp.float32),
                  jax.ShapeDtypeStruct((e, 2 * F), jnp.float32)],
        mesh=mesh,
        scratch_types=[pltpu.VMEM((kg, C), jnp.int32),
                       pltpu.VMEM((kg, C), jnp.int32),
                       pltpu.VMEM((C, 2 * F), jnp.float32),
                       pltpu.VMEM((C, 2 * F), jnp.float32),
                       pltpu.SemaphoreType.DMA,
                       pltpu.SemaphoreType.DMA],
    )
    def gather_k(ab_hbm, row_hbm, col_hbm, ga_hbm, gb_hbm,
                 idxr, idxc, bufa, bufb, sema, semb):
        w = lax.axis_index("s") * NC + lax.axis_index("c")
        base = w * kg
        pltpu.sync_copy(row_hbm.at[pl.ds(base, kg)], idxr)
        pltpu.sync_copy(col_hbm.at[pl.ds(base, kg)], idxc)

        def body(j, carry):
            chunk = base + j

            @pl.when(chunk < nchunk)
            def _():
                ca = pltpu.async_copy(ab_hbm.at[idxr.at[j]], bufa, sema)
                cb = pltpu.async_copy(ab_hbm.at[idxc.at[j]], bufb, semb)
                ca.wait()
                cb.wait()
                pltpu.sync_copy(bufa, ga_hbm.at[pl.ds(chunk * C, C)])
                pltpu.sync_copy(bufb, gb_hbm.at[pl.ds(chunk * C, C)])

            return carry

        lax.fori_loop(0, kg, body, 0)

    return gather_k


KB = 24


@functools.lru_cache(maxsize=None)
def _make_scatter(n, e):
    nh = n // 2
    na = _ceil_to(nh + 1, NS * C)
    rps = na // NS
    npc = rps // C
    nchunk = e // C
    ko = -(-(-(-nchunk // NS)) // KB)
    ks = ko * KB
    mesh = plsc.VectorSubcoreMesh(core_axis_name="c", subcore_axis_name="s")

    @functools.partial(
        pl.kernel,
        out_type=jax.ShapeDtypeStruct((NC * na, F), jnp.float32),
        mesh=mesh,
        scratch_types=[pltpu.VMEM((C,), jnp.int32),
                       pltpu.VMEM((C, F), jnp.float32),
                       pltpu.VMEM_SHARED((na, F), jnp.float32)],
    )
    def scatter_k(ec_hbm, colh_hbm, seqs_hbm, zeros_hbm, recv_hbm,
                  idx, buf, acc):
        c = lax.axis_index("c")
        s = lax.axis_index("s")
        base = s * ks
        cbase = (c * (ks * NS) + base) * C
        pltpu.sync_copy(zeros_hbm, buf)
        for t in range(npc):
            pltpu.sync_copy(seqs_hbm.at[pl.ds((s * npc + t) * C, C)], idx)
            pltpu.sync_copy(buf, acc.at[idx])
        plsc.subcore_barrier()

        def body(j, carry):
            chunk = base + j

            @pl.when(chunk < nchunk)
            def _():
                pltpu.sync_copy(colh_hbm.at[pl.ds(cbase + j * C, C)], idx)
                pltpu.sync_copy(ec_hbm.at[pl.ds(chunk * C, C)], buf)
                pltpu.sync_copy(buf, acc.at[idx], add=True)

            return carry

        lax.fori_loop(0, ks, body, 0)
        plsc.subcore_barrier()
        for t in range(npc):
            pltpu.sync_copy(seqs_hbm.at[pl.ds((s * npc + t) * C, C)], idx)
            pltpu.sync_copy(acc.at[idx], buf)
            pltpu.sync_copy(
                buf, recv_hbm.at[pl.ds(c * na + (s * npc + t) * C, C)])

    return scatter_k


def kernel(x, edge_index, edge_attr, params):
    n = x.shape[0]
    e = edge_index.shape[1]
    bn = 1000 if (n // 2) % 1000 == 0 else n // 2
    be = 4000 if e % 4000 == 0 else e
    nh = n // 2
    na = _ceil_to(nh + 1, NS * C)
    nw = NC * NS
    nchunk = e // C
    kg = _ceil_to(-(-nchunk // nw), 8)
    ks = (-(-(-(-nchunk // NS)) // KB)) * KB

    new_order = jnp.argsort(edge_index[1]).reshape(C, nchunk).T.reshape(e)
    row = edge_index[0][new_order]
    col = edge_index[1][new_order]
    edge_attr = edge_attr[new_order]
    pad_g = kg * nw - nchunk
    row2 = jnp.pad(row.reshape(nchunk, C), ((0, pad_g), (0, 0)))
    col2 = jnp.pad(col.reshape(nchunk, C), ((0, pad_g), (0, 0)))
    dummy = nh + jnp.arange(e, dtype=jnp.int32) % (na - nh)
    colh = jnp.stack([jnp.where(col < nh, col, dummy),
                      jnp.where(col >= nh, col - nh, dummy)])
    colh = jnp.pad(colh.reshape(NC, nchunk, C),
                   ((0, 0), (0, ks * NS - nchunk), (0, 0)))
    colh = colh.reshape(NC * ks * NS * C)
    seqs = jnp.arange(na, dtype=jnp.int32)
    zeros = jnp.zeros((C, F), jnp.float32)

    pn = params['node_encoder']
    pe = params['edge_encoder']
    eu = params['edge_update']
    nu = params['node_update']
    pd = params['decoder']
    w1e = eu['W1']
    wsrc, wdst = w1e[0:F], w1e[F:2 * F]
    wec, wee = w1e[2 * F:3 * F], w1e[3 * F:4 * F]
    beu1 = eu['b1'].reshape(1, F)
    w1n = nu['W1']
    wnx, wnenc, wnr = w1n[0:F], w1n[F:2 * F], w1n[2 * F:3 * F]
    bnu1 = nu['b1'].reshape(1, F)

    wsd = jnp.concatenate([wsrc, wdst], axis=1)
    encx, ab, dx = _encode_nodes(
        x, pn['W1'].reshape(1, F), pn['b1'].reshape(1, F), pn['W2'],
        pn['b2'].reshape(1, F), wsd, wnenc, bnu1, bn=bn)
    ec, cee = _encode_edges(
        edge_attr, pe['W1'], pe['b1'].reshape(1, F), pe['W2'],
        pe['b2'].reshape(1, F), wee, beu1, be=be)

    gather = _make_gather(n, e)
    scatter = _make_scatter(n, e)
    we2 = eu['W2']
    be2 = eu['b2'].reshape(1, F)
    wn2 = nu['W2']
    bn2 = nu['b2'].reshape(1, F)

    xc = encx
    for _ in range(STEPS):
        ga2, gb2 = gather(ab, row2, col2)
        ec = _edge_step(ga2, gb2, ec, cee, wec, we2, be2, be=be)
        recv = jax.ops.segment_sum(ec, col, num_segments=n)
        recv2 = jnp.stack([jnp.pad(recv[:nh], ((0, na - nh), (0, 0))),
                           jnp.pad(recv[nh:], ((0, na - nh), (0, 0)))])
        xc, ab = _node_step(recv2, xc, dx, wnx, wnr, wn2, bn2, wsd, bn=bn)

    return _decode(xc, pd['W1'], pd['b1'].reshape(1, F), pd['W2'],
                   pd['b2'].reshape(1, F), pd['W3'], pd['b3'].reshape(1, 1),
                   bn=bn)
